# baseline (speedup 1.0000x reference)
import functools

import jax
import jax.numpy as jnp
from jax.experimental import pallas as pl
from jax.experimental.pallas import tpu as pltpu

N = 10000
E = 640000
C = 128
G = 64


def _layernorm(h, g, b, eps=1e-5):
    m = h.mean(-1, keepdims=True)
    v = ((h - m) ** 2).mean(-1, keepdims=True)
    return (h - m) / jnp.sqrt(v + eps) * g + b


def _batchnorm(h, g, b, eps=1e-5):
    m = h.mean(0)
    v = ((h - m) ** 2).mean(0)
    return (h - m) / jnp.sqrt(v + eps) * g + b


def _head_body(x_ref, x1_ref, x2_ref, x3_ref, f1a_ref, f1b_ref, f1c_ref,
               f1d_ref, F1b_ref, bn1g_ref, bn1b_ref, F2W_ref, F2b_ref,
               bn2g_ref, bn2b_ref, PW_ref, Pb_ref, h2_ref, s_ref):
    h = (x_ref[:] * f1a_ref[:]
         + jnp.dot(x1_ref[:], f1b_ref[:], preferred_element_type=jnp.float32)
         + jnp.dot(x2_ref[:], f1c_ref[:], preferred_element_type=jnp.float32)
         + jnp.dot(x3_ref[:], f1d_ref[:], preferred_element_type=jnp.float32)
         + F1b_ref[:])
    h = jax.nn.relu(h)
    h = _batchnorm(h, bn1g_ref[:], bn1b_ref[:])
    h = jnp.dot(h, F2W_ref[:], preferred_element_type=jnp.float32) + F2b_ref[:]
    h = jax.nn.relu(h)
    h2 = _batchnorm(h, bn2g_ref[:], bn2b_ref[:])
    h2_ref[:] = h2
    s = jnp.dot(h2, PW_ref[:], preferred_element_type=jnp.float32) + Pb_ref[:]
    s_ref[:] = jax.nn.sigmoid(s)


def _head(x, x1, x2, x3, F1W, F1b, bn1_g, bn1_b, F2W, F2b, bn2_g, bn2_b,
          PW, Pb):
    f1a = F1W[0:1, :]
    f1b = F1W[1:1 + C, :]
    f1c = F1W[1 + C:1 + 2 * C, :]
    f1d = F1W[1 + 2 * C:1 + 3 * C, :]
    out = pl.pallas_call(
        _head_body,
        out_shape=(
            jax.ShapeDtypeStruct((N, C), jnp.float32),
            jax.ShapeDtypeStruct((N, 1), jnp.float32),
        ),
    )(x, x1, x2, x3, f1a, f1b, f1c, f1d,
      F1b.reshape(1, -1), bn1_g.reshape(1, -1), bn1_b.reshape(1, -1),
      F2W, F2b.reshape(1, -1), bn2_g.reshape(1, -1), bn2_b.reshape(1, -1),
      PW, Pb.reshape(1, 1))
    return out


def kernel(x, edge_index, edge_attr, batch, W1, b1, ln1_g, ln1_b, e2W, e2b,
           W2, b2, ln2_g, ln2_b, e3W, e3b, W3, b3, ln3_g, ln3_b, F1W, F1b,
           bn1_g, bn1_b, F2W, F2b, bn2_g, bn2_b, PW, Pb, VW, Vb):
    src = edge_index[0]
    dst = edge_index[1]

    msg = jax.nn.relu(x[src, 0] + edge_attr[:, 0])
    agg = jax.ops.segment_sum(msg, dst, num_segments=N)
    h = x[:, 0] + agg
    x1 = _layernorm(jax.nn.relu(h[:, None] * W1[0][None, :] + b1),
                    ln1_g, ln1_b)

    add2 = edge_attr[:, 0:1] * e2W[0][None, :] + e2b
    msg = jax.nn.relu(x1[src] + add2)
    agg = jax.ops.segment_sum(msg, dst, num_segments=N)
    x2 = _layernorm(jax.nn.relu((x1 + agg) @ W2 + b2), ln2_g, ln2_b)

    add3 = edge_attr[:, 0:1] * e3W[0][None, :] + e3b
    msg = jax.nn.relu(x2[src] + add3)
    agg = jax.ops.segment_sum(msg, dst, num_segments=N)
    x3 = _layernorm(jax.nn.relu((x2 + agg) @ W3 + b3), ln3_g, ln3_b)

    h2, s = _head(x, x1, x2, x3, F1W, F1b, bn1_g, bn1_b, F2W, F2b,
                  bn2_g, bn2_b, PW, Pb)

    edge_probs = s[src, 0]
    counts = jax.ops.segment_sum(jnp.ones((N,), dtype=h2.dtype), batch,
                                 num_segments=G)
    pooled = (jax.ops.segment_sum(h2, batch, num_segments=G)
              / jnp.clip(counts, 1.0)[:, None])
    value = jnp.tanh(pooled @ VW + Vb).squeeze(-1)
    return (edge_probs, value)


# speedup vs baseline: 1.4069x; 1.4069x over previous
import functools

import jax
import jax.numpy as jnp
from jax import lax
from jax.experimental import pallas as pl
from jax.experimental.pallas import tpu as pltpu
from jax.experimental.pallas import tpu_sc as plsc

N = 10000
E = 640000
C = 128
G = 64

NC = 2
NS = 16
L = 16
NW = NC * NS
EPW = E // NW
CHUNK = 80
NCHUNK = EPW // CHUNK
NP = 10240
RPT = NP // NS


def _layernorm(h, g, b, eps=1e-5):
    m = h.mean(-1, keepdims=True)
    v = ((h - m) ** 2).mean(-1, keepdims=True)
    return (h - m) / jnp.sqrt(v + eps) * g + b


def _batchnorm(h, g, b, eps=1e-5):
    m = h.mean(0)
    v = ((h - m) ** 2).mean(0)
    return (h - m) / jnp.sqrt(v + eps) * g + b


def _edge_pass_body(xp_hbm, src_hbm, dst_hbm, ea_hbm, w_hbm, b_hbm, zeros_hbm,
                    out_hbm, src_v, dst_v, ea_v, rows_v, w_v, b_v, acc, sem):
    c = lax.axis_index("c")
    s = lax.axis_index("s")
    wid = s * NC + c

    pltpu.sync_copy(zeros_hbm.at[pl.ds(s * RPT, RPT)],
                    acc.at[pl.ds(s * RPT, RPT)])
    pltpu.sync_copy(w_hbm, w_v)
    pltpu.sync_copy(b_hbm, b_v)
    plsc.subcore_barrier()

    wvecs = [w_v[pl.ds(L * i, L)] for i in range(C // L)]
    bvecs = [b_v[pl.ds(L * i, L)] for i in range(C // L)]
    base0 = wid * EPW

    def chunk_body(ci, carry):
        base = base0 + ci * CHUNK
        pltpu.sync_copy(src_hbm.at[pl.ds(base, CHUNK)], src_v)
        pltpu.sync_copy(dst_hbm.at[pl.ds(base, CHUNK)], dst_v)
        pltpu.sync_copy(ea_hbm.at[pl.ds(base, CHUNK)], ea_v)
        pltpu.async_copy(xp_hbm.at[src_v], rows_v, sem).wait()

        def group_body(k, carry2):
            ea16 = ea_v[pl.ds(L * k, L)]
            for jj in range(L):
                j = L * k + jj
                alpha = jnp.broadcast_to(ea16[jj], (L,))
                for i in range(C // L):
                    v = rows_v[j, pl.ds(L * i, L)]
                    v = jnp.maximum(v + (alpha * wvecs[i] + bvecs[i]), 0.0)
                    rows_v[j, pl.ds(L * i, L)] = v
            return carry2

        lax.fori_loop(0, CHUNK // L, group_body, 0, unroll=False)
        pltpu.sync_copy(rows_v, acc.at[dst_v], add=True)
        return carry

    lax.fori_loop(0, NCHUNK, chunk_body, 0, unroll=False)
    plsc.subcore_barrier()
    pltpu.sync_copy(acc.at[pl.ds(s * RPT, RPT)],
                    out_hbm.at[c].at[pl.ds(s * RPT, RPT)])


@jax.jit
def _sc_edge_pass(xp, src, dst, ea, w, b, zeros):
    mesh = plsc.VectorSubcoreMesh(core_axis_name="c", subcore_axis_name="s",
                                  num_cores=NC, num_subcores=NS)
    f = pl.kernel(
        _edge_pass_body,
        out_type=jax.ShapeDtypeStruct((NC, NP, C), jnp.float32),
        mesh=mesh,
        scratch_types=[
            pltpu.VMEM((CHUNK,), jnp.int32),
            pltpu.VMEM((CHUNK,), jnp.int32),
            pltpu.VMEM((CHUNK,), jnp.float32),
            pltpu.VMEM((CHUNK, C), jnp.float32),
            pltpu.VMEM((C,), jnp.float32),
            pltpu.VMEM((C,), jnp.float32),
            pltpu.VMEM_SHARED((NP, C), jnp.float32),
            pltpu.SemaphoreType.DMA,
        ],
    )
    return f(xp, src, dst, ea, w, b, zeros)


def _head_body(x_ref, x1_ref, x2_ref, x3_ref, f1a_ref, f1b_ref, f1c_ref,
               f1d_ref, F1b_ref, bn1g_ref, bn1b_ref, F2W_ref, F2b_ref,
               bn2g_ref, bn2b_ref, PW_ref, Pb_ref, h2_ref, s_ref):
    h = (x_ref[:] * f1a_ref[:]
         + jnp.dot(x1_ref[:], f1b_ref[:], preferred_element_type=jnp.float32)
         + jnp.dot(x2_ref[:], f1c_ref[:], preferred_element_type=jnp.float32)
         + jnp.dot(x3_ref[:], f1d_ref[:], preferred_element_type=jnp.float32)
         + F1b_ref[:])
    h = jax.nn.relu(h)
    h = _batchnorm(h, bn1g_ref[:], bn1b_ref[:])
    h = jnp.dot(h, F2W_ref[:], preferred_element_type=jnp.float32) + F2b_ref[:]
    h = jax.nn.relu(h)
    h2 = _batchnorm(h, bn2g_ref[:], bn2b_ref[:])
    h2_ref[:] = h2
    s = jnp.dot(h2, PW_ref[:], preferred_element_type=jnp.float32) + Pb_ref[:]
    s_ref[:] = jax.nn.sigmoid(s)


def _head(x, x1, x2, x3, F1W, F1b, bn1_g, bn1_b, F2W, F2b, bn2_g, bn2_b,
          PW, Pb):
    f1a = F1W[0:1, :]
    f1b = F1W[1:1 + C, :]
    f1c = F1W[1 + C:1 + 2 * C, :]
    f1d = F1W[1 + 2 * C:1 + 3 * C, :]
    return pl.pallas_call(
        _head_body,
        out_shape=(
            jax.ShapeDtypeStruct((N, C), jnp.float32),
            jax.ShapeDtypeStruct((N, 1), jnp.float32),
        ),
    )(x, x1, x2, x3, f1a, f1b, f1c, f1d,
      F1b.reshape(1, -1), bn1_g.reshape(1, -1), bn1_b.reshape(1, -1),
      F2W, F2b.reshape(1, -1), bn2_g.reshape(1, -1), bn2_b.reshape(1, -1),
      PW, Pb.reshape(1, 1))


def _dense_body(xp_ref, p0_ref, p1_ref, W_ref, b_ref, lg_ref, lb_ref, o_ref):
    h = xp_ref[:] + p0_ref[:] + p1_ref[:]
    h = jnp.dot(h, W_ref[:], preferred_element_type=jnp.float32) + b_ref[:]
    h = jax.nn.relu(h)
    o_ref[:] = _layernorm(h, lg_ref[:], lb_ref[:])


def _dense(xp, parts, W, b, lg, lb):
    return pl.pallas_call(
        _dense_body,
        out_shape=jax.ShapeDtypeStruct((N, C), jnp.float32),
    )(xp, parts[0, :N], parts[1, :N], W, b.reshape(1, -1),
      lg.reshape(1, -1), lb.reshape(1, -1))


def kernel(x, edge_index, edge_attr, batch, W1, b1, ln1_g, ln1_b, e2W, e2b,
           W2, b2, ln2_g, ln2_b, e3W, e3b, W3, b3, ln3_g, ln3_b, F1W, F1b,
           bn1_g, bn1_b, F2W, F2b, bn2_g, bn2_b, PW, Pb, VW, Vb):
    src = edge_index[0].astype(jnp.int32)
    dst = edge_index[1].astype(jnp.int32)
    ea = edge_attr[:, 0]
    zeros = jnp.zeros((NP, C), jnp.float32)

    msg = jax.nn.relu(x[src, 0] + ea)
    agg = jax.ops.segment_sum(msg, dst, num_segments=N)
    h = x[:, 0] + agg
    x1 = _layernorm(jax.nn.relu(h[:, None] * W1[0][None, :] + b1),
                    ln1_g, ln1_b)

    parts = _sc_edge_pass(x1, src, dst, ea, e2W[0], e2b, zeros)
    x2 = _dense(x1, parts, W2, b2, ln2_g, ln2_b)

    parts = _sc_edge_pass(x2, src, dst, ea, e3W[0], e3b, zeros)
    x3 = _dense(x2, parts, W3, b3, ln3_g, ln3_b)

    h2, s = _head(x, x1, x2, x3, F1W, F1b, bn1_g, bn1_b, F2W, F2b,
                  bn2_g, bn2_b, PW, Pb)

    edge_probs = s[src, 0]
    counts = jax.ops.segment_sum(jnp.ones((N,), dtype=h2.dtype), batch,
                                 num_segments=G)
    pooled = (jax.ops.segment_sum(h2, batch, num_segments=G)
              / jnp.clip(counts, 1.0)[:, None])
    value = jnp.tanh(pooled @ VW + Vb).squeeze(-1)
    return (edge_probs, value)


# speedup vs baseline: 15.8185x; 11.2436x over previous
import functools

import jax
import jax.numpy as jnp
from jax import lax
from jax.experimental import pallas as pl
from jax.experimental.pallas import tpu as pltpu
from jax.experimental.pallas import tpu_sc as plsc

N = 10000
E = 640000
C = 128
G = 64

NC = 2
NS = 16
L = 16
NW = NC * NS
EPW = E // NW
CHUNK = 80
NCHUNK = EPW // CHUNK
SUB = 50
NSUP = NCHUNK // SUB
NP = 10240
RPT = NP // NS


def _layernorm(h, g, b, eps=1e-5):
    m = h.mean(-1, keepdims=True)
    v = ((h - m) ** 2).mean(-1, keepdims=True)
    return (h - m) / jnp.sqrt(v + eps) * g + b


def _batchnorm(h, g, b, eps=1e-5):
    m = h.mean(0)
    v = ((h - m) ** 2).mean(0)
    return (h - m) / jnp.sqrt(v + eps) * g + b


def _edge_pass_body(cw, xp_hbm, src_hbm, dst_hbm, ea_hbm, w_hbm, b_hbm,
                    zeros_hbm, out_hbm, src_v, dst_v, ea_v, rows0, rows1,
                    w_v, b_v, acc, gsem0, gsem1, ssem0, ssem1):
    c = lax.axis_index("c")
    s = lax.axis_index("s")
    wid = s * NC + c

    pltpu.sync_copy(zeros_hbm.at[pl.ds(s * RPT, RPT)],
                    acc.at[pl.ds(s * RPT, RPT)])
    pltpu.sync_copy(w_hbm, w_v)
    pltpu.sync_copy(b_hbm, b_v)
    plsc.subcore_barrier()

    wvecs = [w_v[pl.ds(L * i, L)] for i in range(cw // L)]
    bvecs = [b_v[pl.ds(L * i, L)] for i in range(cw // L)]

    def compute(rows_v, ci):
        def group_body(k, carry2):
            ea16 = ea_v[ci, pl.ds(L * k, L)]
            for jj in range(L):
                j = L * k + jj
                alpha = jnp.broadcast_to(ea16[jj], (L,))
                for i in range(cw // L):
                    v = rows_v[j, pl.ds(L * i, L)]
                    v = jnp.maximum(v + (alpha * wvecs[i] + bvecs[i]), 0.0)
                    rows_v[j, pl.ds(L * i, L)] = v
            return carry2

        lax.fori_loop(0, CHUNK // L, group_body, 0, unroll=False)

    bufs = ((rows0, gsem0, ssem0), (rows1, gsem1, ssem1))

    def sup_body(sup, carry):
        pltpu.sync_copy(src_hbm.at[wid, sup], src_v)
        pltpu.sync_copy(dst_hbm.at[wid, sup], dst_v)
        pltpu.sync_copy(ea_hbm.at[wid, sup], ea_v)
        pltpu.make_async_copy(xp_hbm.at[src_v.at[0]], rows0, gsem0).start()

        def pair_body(p, carry2):
            for bi, (rows_v, gsem, ssem) in enumerate(bufs):
                ci = 2 * p + bi
                nci = ci + 1
                ob = bufs[(bi + 1) % 2]

                @pl.when(nci < SUB)
                def _():
                    @pl.when(nci >= 2)
                    def _():
                        pltpu.make_async_copy(ob[0], acc.at[dst_v.at[0]],
                                              ob[2]).wait()
                    pltpu.make_async_copy(xp_hbm.at[src_v.at[nci]], ob[0],
                                          ob[1]).start()

                pltpu.make_async_copy(xp_hbm.at[src_v.at[ci]], rows_v,
                                      gsem).wait()
                compute(rows_v, ci)
                pltpu.make_async_copy(rows_v, acc.at[dst_v.at[ci]],
                                      ssem).start(add=True)
            return carry2

        lax.fori_loop(0, SUB // 2, pair_body, 0, unroll=False)
        for rows_v, gsem, ssem in bufs:
            pltpu.make_async_copy(rows_v, acc.at[dst_v.at[0]], ssem).wait()
        return carry

    lax.fori_loop(0, NSUP, sup_body, 0, unroll=False)
    plsc.subcore_barrier()
    pltpu.sync_copy(acc.at[pl.ds(s * RPT, RPT)],
                    out_hbm.at[c].at[pl.ds(s * RPT, RPT)])


def _sc_edge_pass(xp, src3, dst3, ea3, w, b, zeros, cw):
    mesh = plsc.VectorSubcoreMesh(core_axis_name="c", subcore_axis_name="s",
                                  num_cores=NC, num_subcores=NS)
    f = pl.kernel(
        functools.partial(_edge_pass_body, cw),
        out_type=jax.ShapeDtypeStruct((NC, NP, cw), jnp.float32),
        mesh=mesh,
        scratch_types=[
            pltpu.VMEM((SUB, CHUNK), jnp.int32),
            pltpu.VMEM((SUB, CHUNK), jnp.int32),
            pltpu.VMEM((SUB, CHUNK), jnp.float32),
            pltpu.VMEM((CHUNK, cw), jnp.float32),
            pltpu.VMEM((CHUNK, cw), jnp.float32),
            pltpu.VMEM((cw,), jnp.float32),
            pltpu.VMEM((cw,), jnp.float32),
            pltpu.VMEM_SHARED((NP, cw), jnp.float32),
            pltpu.SemaphoreType.DMA,
            pltpu.SemaphoreType.DMA,
            pltpu.SemaphoreType.DMA,
            pltpu.SemaphoreType.DMA,
        ],
    )
    return f(xp, src3, dst3, ea3, w, b, zeros)


def _gather_body(tab_hbm, idx_hbm, out_hbm, idx_v, rows0, rows1,
                 gsem0, gsem1, ssem0, ssem1):
    c = lax.axis_index("c")
    s = lax.axis_index("s")
    wid = s * NC + c
    base0 = wid * EPW
    bufs = ((rows0, gsem0, ssem0), (rows1, gsem1, ssem1))

    def sup_body(sup, carry):
        pltpu.sync_copy(idx_hbm.at[wid, sup], idx_v)
        pltpu.make_async_copy(tab_hbm.at[idx_v.at[0]], rows0, gsem0).start()
        sbase = base0 + sup * SUB * CHUNK

        def pair_body(p, carry2):
            for bi, (rows_v, gsem, ssem) in enumerate(bufs):
                ci = 2 * p + bi
                nci = ci + 1
                ob = bufs[(bi + 1) % 2]

                @pl.when(nci < SUB)
                def _():
                    @pl.when(nci >= 2)
                    def _():
                        pltpu.make_async_copy(
                            ob[0], out_hbm.at[pl.ds(sbase, CHUNK)],
                            ob[2]).wait()
                    pltpu.make_async_copy(tab_hbm.at[idx_v.at[nci]], ob[0],
                                          ob[1]).start()

                pltpu.make_async_copy(tab_hbm.at[idx_v.at[ci]], rows_v,
                                      gsem).wait()
                pltpu.make_async_copy(
                    rows_v, out_hbm.at[pl.ds(sbase + ci * CHUNK, CHUNK)],
                    ssem).start()
            return carry2

        lax.fori_loop(0, SUB // 2, pair_body, 0, unroll=False)
        for rows_v, gsem, ssem in bufs:
            pltpu.make_async_copy(rows_v, out_hbm.at[pl.ds(sbase, CHUNK)],
                                  ssem).wait()
        return carry

    lax.fori_loop(0, NSUP, sup_body, 0, unroll=False)


def _sc_gather(tab, idx3):
    mesh = plsc.VectorSubcoreMesh(core_axis_name="c", subcore_axis_name="s",
                                  num_cores=NC, num_subcores=NS)
    f = pl.kernel(
        _gather_body,
        out_type=jax.ShapeDtypeStruct((E, C), jnp.float32),
        mesh=mesh,
        scratch_types=[
            pltpu.VMEM((SUB, CHUNK), jnp.int32),
            pltpu.VMEM((CHUNK, C), jnp.float32),
            pltpu.VMEM((CHUNK, C), jnp.float32),
            pltpu.SemaphoreType.DMA,
            pltpu.SemaphoreType.DMA,
            pltpu.SemaphoreType.DMA,
            pltpu.SemaphoreType.DMA,
        ],
    )
    return f(tab, idx3)


def _head_body(x_ref, x1_ref, x2_ref, x3_ref, batch_ref, f1a_ref, f1b_ref,
               f1c_ref, f1d_ref, F1b_ref, bn1g_ref, bn1b_ref, F2W_ref,
               F2b_ref, bn2g_ref, bn2b_ref, PW_ref, Pb_ref, VW_ref, Vb_ref,
               s_ref, val_ref):
    h = (x_ref[:] * f1a_ref[:]
         + jnp.dot(x1_ref[:], f1b_ref[:], preferred_element_type=jnp.float32)
         + jnp.dot(x2_ref[:], f1c_ref[:], preferred_element_type=jnp.float32)
         + jnp.dot(x3_ref[:], f1d_ref[:], preferred_element_type=jnp.float32)
         + F1b_ref[:])
    h = jax.nn.relu(h)
    h = _batchnorm(h, bn1g_ref[:], bn1b_ref[:])
    h = jnp.dot(h, F2W_ref[:], preferred_element_type=jnp.float32) + F2b_ref[:]
    h = jax.nn.relu(h)
    h2 = _batchnorm(h, bn2g_ref[:], bn2b_ref[:])
    s = jnp.dot(h2, PW_ref[:], preferred_element_type=jnp.float32) + Pb_ref[:]
    s_ref[:] = jax.nn.sigmoid(s)
    gids = lax.broadcasted_iota(jnp.int32, (G, N), 0)
    mask = (gids == batch_ref[:]).astype(jnp.float32)
    counts = jnp.sum(mask, axis=1, keepdims=True)
    pooled = (jnp.dot(mask, h2, preferred_element_type=jnp.float32)
              / jnp.maximum(counts, 1.0))
    val = jnp.dot(pooled, VW_ref[:], preferred_element_type=jnp.float32)
    val_ref[:] = jnp.tanh(val + Vb_ref[:])


def _head(x, x1, x2, x3, batch, F1W, F1b, bn1_g, bn1_b, F2W, F2b, bn2_g,
          bn2_b, PW, Pb, VW, Vb):
    f1a = F1W[0:1, :]
    f1b = F1W[1:1 + C, :]
    f1c = F1W[1 + C:1 + 2 * C, :]
    f1d = F1W[1 + 2 * C:1 + 3 * C, :]
    return pl.pallas_call(
        _head_body,
        out_shape=(
            jax.ShapeDtypeStruct((N, 1), jnp.float32),
            jax.ShapeDtypeStruct((G, 1), jnp.float32),
        ),
    )(x, x1, x2, x3, batch.reshape(1, N).astype(jnp.int32),
      f1a, f1b, f1c, f1d,
      F1b.reshape(1, -1), bn1_g.reshape(1, -1), bn1_b.reshape(1, -1),
      F2W, F2b.reshape(1, -1), bn2_g.reshape(1, -1), bn2_b.reshape(1, -1),
      PW, Pb.reshape(1, 1), VW, Vb.reshape(1, 1))


def _dense_body(xp_ref, p0_ref, p1_ref, W_ref, b_ref, lg_ref, lb_ref, o_ref):
    h = xp_ref[:] + p0_ref[:] + p1_ref[:]
    h = jnp.dot(h, W_ref[:], preferred_element_type=jnp.float32) + b_ref[:]
    h = jax.nn.relu(h)
    o_ref[:] = _layernorm(h, lg_ref[:], lb_ref[:])


def _dense(xp, parts, W, b, lg, lb):
    return pl.pallas_call(
        _dense_body,
        out_shape=jax.ShapeDtypeStruct((N, C), jnp.float32),
    )(xp, parts[0, :N], parts[1, :N], W, b.reshape(1, -1),
      lg.reshape(1, -1), lb.reshape(1, -1))


def kernel(x, edge_index, edge_attr, batch, W1, b1, ln1_g, ln1_b, e2W, e2b,
           W2, b2, ln2_g, ln2_b, e3W, e3b, W3, b3, ln3_g, ln3_b, F1W, F1b,
           bn1_g, bn1_b, F2W, F2b, bn2_g, bn2_b, PW, Pb, VW, Vb):
    src3 = edge_index[0].astype(jnp.int32).reshape(NW, NSUP, SUB, CHUNK)
    dst3 = edge_index[1].astype(jnp.int32).reshape(NW, NSUP, SUB, CHUNK)
    ea3 = edge_attr[:, 0].reshape(NW, NSUP, SUB, CHUNK)
    zeros = jnp.zeros((NP, C), jnp.float32)

    xrep = jnp.broadcast_to(x, (N, C))
    parts1 = _sc_edge_pass(xrep, src3, dst3, ea3, jnp.ones((C,), jnp.float32),
                           jnp.zeros((C,), jnp.float32), zeros, C)
    agg1 = parts1[0, :N, 0] + parts1[1, :N, 0]
    h = x[:, 0] + agg1
    x1 = _layernorm(jax.nn.relu(h[:, None] * W1[0][None, :] + b1),
                    ln1_g, ln1_b)

    parts = _sc_edge_pass(x1, src3, dst3, ea3, e2W[0], e2b, zeros, C)
    x2 = _dense(x1, parts, W2, b2, ln2_g, ln2_b)

    parts = _sc_edge_pass(x2, src3, dst3, ea3, e3W[0], e3b, zeros, C)
    x3 = _dense(x2, parts, W3, b3, ln3_g, ln3_b)

    s, val = _head(x, x1, x2, x3, batch, F1W, F1b, bn1_g, bn1_b, F2W, F2b,
                   bn2_g, bn2_b, PW, Pb, VW, Vb)

    srep = jnp.broadcast_to(s, (N, C))
    edge_probs = _sc_gather(srep, src3)[:, 0]
    value = val[:, 0]
    return (edge_probs, value)
